# baseline (speedup 1.0000x reference)
import jax
import jax.numpy as jnp
from jax.experimental import pallas as pl


def _split_body(x_ref, arm_ref, other_ref):
    x = x_ref[...]
    arm_ref[...] = x[..., 5:11]
    other_ref[...] = jnp.concatenate([x[..., 0:5], x[..., 11:17]], axis=-1)


def kernel(x):
    B, C, T, K = x.shape
    CB = 8
    grid = (B, C // CB)
    out_shapes = (
        jax.ShapeDtypeStruct((B, C, T, 6), x.dtype),
        jax.ShapeDtypeStruct((B, C, T, 11), x.dtype),
    )
    arm, other = pl.pallas_call(
        _split_body,
        grid=grid,
        in_specs=[pl.BlockSpec((1, CB, T, K), lambda i, j: (i, j, 0, 0))],
        out_specs=(
            pl.BlockSpec((1, CB, T, 6), lambda i, j: (i, j, 0, 0)),
            pl.BlockSpec((1, CB, T, 11), lambda i, j: (i, j, 0, 0)),
        ),
        out_shape=out_shapes,
    )(x)
    return (arm, other)
